# initial kernel scaffold (unmeasured)
import jax
import jax.numpy as jnp
from jax import lax
from jax.experimental import pallas as pl
from jax.experimental.pallas import tpu as pltpu


def kernel(
    x,
):
    def body(*refs):
        pass

    out_shape = jax.ShapeDtypeStruct(..., jnp.float32)
    return pl.pallas_call(body, out_shape=out_shape)(...)


# baseline (device time: 28248 ns/iter reference)
import jax
import jax.numpy as jnp
from jax import lax
from jax.experimental import pallas as pl
from jax.experimental.pallas import tpu as pltpu

N_DEV = 4
M = 512
CH = 512


def kernel(x):
    def body(x_ref, out_ref, send_buf, recv_buf, send_sems, recv_sems):
        my = lax.axis_index("i")
        left = lax.rem(my + N_DEV - 1, N_DEV)
        right = lax.rem(my + 1, N_DEV)

        barrier_sem = pltpu.get_barrier_semaphore()
        for nbr in (left, right):
            pl.semaphore_signal(
                barrier_sem, inc=1,
                device_id=(nbr,), device_id_type=pl.DeviceIdType.MESH,
            )
        pl.semaphore_wait(barrier_sem, 2)

        c0 = lax.rem(my + 3, N_DEV)
        send_buf[0, :, :] = x_ref[0, :, pl.ds(c0 * CH, CH)].astype(jnp.bfloat16)

        for h in range(N_DEV - 1):
            rdma = pltpu.make_async_remote_copy(
                src_ref=send_buf.at[h],
                dst_ref=recv_buf.at[h],
                send_sem=send_sems.at[h],
                recv_sem=recv_sems.at[h],
                device_id=(right,),
                device_id_type=pl.DeviceIdType.MESH,
            )
            rdma.start()
            rdma.wait()

            cj = lax.rem(my + 2 - h, N_DEV)
            contrib = x_ref[0, :, pl.ds(cj * CH, CH)].astype(jnp.bfloat16)
            if h < N_DEV - 2:
                send_buf[h + 1, :, :] = recv_buf[h, :, :] + contrib
            else:
                out_ref[:, :] = recv_buf[h, :, :] + contrib

    return pl.pallas_call(
        body,
        out_shape=jax.ShapeDtypeStruct((M, CH), jnp.bfloat16),
        in_specs=[pl.BlockSpec(memory_space=pltpu.VMEM)],
        out_specs=pl.BlockSpec(memory_space=pltpu.VMEM),
        scratch_shapes=[
            pltpu.VMEM((N_DEV - 1, M, CH), jnp.bfloat16),
            pltpu.VMEM((N_DEV - 1, M, CH), jnp.bfloat16),
            pltpu.SemaphoreType.DMA((N_DEV - 1,)),
            pltpu.SemaphoreType.DMA((N_DEV - 1,)),
        ],
        compiler_params=pltpu.CompilerParams(collective_id=0),
    )(x)


# device time: 17220 ns/iter; 1.6404x vs baseline; 1.6404x over previous
import jax
import jax.numpy as jnp
from jax import lax
from jax.experimental import pallas as pl
from jax.experimental.pallas import tpu as pltpu

N_DEV = 4
M = 512
CH = 512
HW = CH // 2
P = 2
SW = HW // P
BF16 = jnp.bfloat16


def kernel(x):
    def body(x_ref, out_ref, sb_r, rb_r, sb_l, rb_l, ss_r, rs_r, ss_l, rs_l):
        my = lax.axis_index("i")
        left = lax.rem(my + N_DEV - 1, N_DEV)
        right = lax.rem(my + 1, N_DEV)

        barrier_sem = pltpu.get_barrier_semaphore()
        for nbr in (left, right):
            pl.semaphore_signal(
                barrier_sem, inc=1,
                device_id=(nbr,), device_id_type=pl.DeviceIdType.MESH,
            )
        pl.semaphore_wait(barrier_sem, 2)

        def mk(dirn, h, s):
            sb, rb, ss, rs = (
                (sb_r, rb_r, ss_r, rs_r) if dirn == 0 else (sb_l, rb_l, ss_l, rs_l)
            )
            tgt = right if dirn == 0 else left
            return pltpu.make_async_remote_copy(
                src_ref=sb.at[h, :, pl.ds(s * SW, SW)],
                dst_ref=rb.at[h, :, pl.ds(s * SW, SW)],
                send_sem=ss.at[h, s],
                recv_sem=rs.at[h, s],
                device_id=(tgt,),
                device_id_type=pl.DeviceIdType.MESH,
            )

        def contrib(dirn, h, s):
            cj = lax.rem(my + 2 + (h if dirn else -h), N_DEV)
            col = cj * CH + (HW if dirn else 0) + s * SW
            return x_ref[0, :, pl.ds(col, SW)].astype(BF16)

        cr0 = lax.rem(my + 3, N_DEV)
        cl0 = lax.rem(my + 1, N_DEV)
        started = {}
        for s in range(P):
            sb_r[0, :, pl.ds(s * SW, SW)] = x_ref[
                0, :, pl.ds(cr0 * CH + s * SW, SW)
            ].astype(BF16)
            started[(0, 0, s)] = mk(0, 0, s)
            started[(0, 0, s)].start()
            sb_l[0, :, pl.ds(s * SW, SW)] = x_ref[
                0, :, pl.ds(cl0 * CH + HW + s * SW, SW)
            ].astype(BF16)
            started[(1, 0, s)] = mk(1, 0, s)
            started[(1, 0, s)].start()

        for h in range(N_DEV - 1):
            for s in range(P):
                for dirn in (0, 1):
                    sb, rb = (sb_r, rb_r) if dirn == 0 else (sb_l, rb_l)
                    started[(dirn, h, s)].wait_recv()
                    acc = rb[h, :, pl.ds(s * SW, SW)] + contrib(dirn, h, s)
                    if h < N_DEV - 2:
                        sb[h + 1, :, pl.ds(s * SW, SW)] = acc
                        started[(dirn, h + 1, s)] = mk(dirn, h + 1, s)
                        started[(dirn, h + 1, s)].start()
                    else:
                        col = (HW if dirn else 0) + s * SW
                        out_ref[:, pl.ds(col, SW)] = acc

        for r in started.values():
            r.wait_send()

    return pl.pallas_call(
        body,
        out_shape=jax.ShapeDtypeStruct((M, CH), BF16),
        in_specs=[pl.BlockSpec(memory_space=pltpu.VMEM)],
        out_specs=pl.BlockSpec(memory_space=pltpu.VMEM),
        scratch_shapes=[
            pltpu.VMEM((N_DEV - 1, M, HW), BF16),
            pltpu.VMEM((N_DEV - 1, M, HW), BF16),
            pltpu.VMEM((N_DEV - 1, M, HW), BF16),
            pltpu.VMEM((N_DEV - 1, M, HW), BF16),
            pltpu.SemaphoreType.DMA((N_DEV - 1, P)),
            pltpu.SemaphoreType.DMA((N_DEV - 1, P)),
            pltpu.SemaphoreType.DMA((N_DEV - 1, P)),
            pltpu.SemaphoreType.DMA((N_DEV - 1, P)),
        ],
        compiler_params=pltpu.CompilerParams(collective_id=0),
    )(x)
